# baseline (speedup 1.0000x reference)
import functools

import jax
import jax.numpy as jnp
from jax import lax
from jax.experimental import pallas as pl
from jax.experimental.pallas import tpu as pltpu
from jax.experimental.pallas import tpu_sc as plsc

B, S, D, W = 4, 8192, 768, 4096
NC, NS, L = 2, 16, 16
DSL = 128
N_SLICES = D // DSL
CHUNK = 64
TOK_PER_TILE = S // NS
N_CHUNKS = TOK_PER_TILE // CHUNK
W_PER_TILE = W // NS
WBLK = 32
N_WBLK = W_PER_TILE // WBLK
B_PER_SC = B // NC
N_ROUNDS = B_PER_SC * N_SLICES


NSLOT = 4


def _pool_body(emb_hbm, ids_hbm, out_hbm,
               ids_all_v, tok2_v, stage4_v, cw_v, zbuf_v,
               acc_sh, cnt_sh,
               gsems, ssems, lsems, wsems, clsems, zsem, csem):
    c = lax.axis_index("c")
    s = lax.axis_index("s")

    zvec = jnp.zeros((L,), jnp.float32)
    ovec = jnp.ones((L,), jnp.float32)

    def _init_row(i, _):
        for j in range(DSL // L):
            zbuf_v[i, pl.ds(j * L, L)] = zvec
        return 0
    lax.fori_loop(0, WBLK, _init_row, 0)

    w_base = s * W_PER_TILE

    zeros0 = []
    for blk in range(N_WBLK):
        zeros0.append(pltpu.async_copy(
            zbuf_v, acc_sh.at[pl.ds(w_base + blk * WBLK, WBLK)], zsem))
        zeros0.append(pltpu.async_copy(
            zbuf_v, cnt_sh.at[pl.ds(w_base + blk * WBLK, WBLK)], csem))
    for d in zeros0:
        d.wait()

    ids_rows = [ids_all_v.at[k] for k in range(N_CHUNKS)]
    tok_bufs = [tok2_v.at[i] for i in range(NSLOT)]
    g_pend = [[] for _ in range(NSLOT)]

    def _round_params(r):
        return c * B_PER_SC + r // N_SLICES, (r % N_SLICES) * DSL

    def _fire_gather(r, k):
        b_, doff_ = _round_params(r)
        slot = k % NSLOT
        t0 = s * TOK_PER_TILE + k * CHUNK
        g_pend[slot].append(pltpu.async_copy(
            emb_hbm.at[b_, pl.ds(t0, CHUNK), pl.ds(doff_, DSL)],
            tok_bufs[slot], gsems[slot]))

    for k in range(NSLOT - 1):
        _fire_gather(0, k)

    plsc.subcore_barrier()

    for r in range(N_ROUNDS):
        b, doff = _round_params(r)
        dslice = r % N_SLICES
        first_slice = dslice == 0
        last_slice = dslice == N_SLICES - 1

        if first_slice:
            i_pend = []
            for k in range(N_CHUNKS):
                t0 = s * TOK_PER_TILE + k * CHUNK
                i_pend.append(pltpu.async_copy(
                    ids_hbm.at[b, pl.ds(t0, CHUNK)], ids_rows[k], csem))
            for d in i_pend:
                d.wait()
            def _init_ones(i, _):
                for j in range(DSL // L):
                    cw_v[i, pl.ds(j * L, L)] = ovec
                return 0
            lax.fori_loop(0, CHUNK, _init_ones, 0)

        s_pend = [[] for _ in range(NSLOT)]
        c_pend = []
        for k in range(N_CHUNKS):
            slot = k % NSLOT
            for d in g_pend[slot]:
                d.wait()
            g_pend[slot] = []
            s_pend[slot].append(pltpu.async_copy(
                tok_bufs[slot], acc_sh.at[ids_rows[k]], ssems[slot],
                add=True))
            if first_slice:
                c_pend.append(pltpu.async_copy(
                    cw_v, cnt_sh.at[ids_rows[k]], csem, add=True))
            nxt = k + NSLOT - 1
            if nxt < N_CHUNKS:
                nslot = nxt % NSLOT
                for d in s_pend[nslot]:
                    d.wait()
                s_pend[nslot] = []
                _fire_gather(r, nxt)
        for slot in range(NSLOT):
            for d in s_pend[slot]:
                d.wait()
        for d in c_pend:
            d.wait()

        if r + 1 < N_ROUNDS:
            for k in range(NSLOT - 1):
                _fire_gather(r + 1, k)

        plsc.subcore_barrier()

        SSLOT = 4
        stage_bufs = [stage4_v.at[i] for i in range(SSLOT)]
        cnt_bufs = [cw_v.at[pl.ds(0, WBLK)], cw_v.at[pl.ds(WBLK, WBLK)]]
        l_pend = [[] for _ in range(SSLOT)]
        c_pendl = [[], []]
        w_pend = [[] for _ in range(SSLOT)]
        z_pend = []

        def _fire_stage_load(blk):
            slot = blk % SSLOT
            w0 = w_base + blk * WBLK
            l_pend[slot].append(pltpu.async_copy(
                acc_sh.at[pl.ds(w0, WBLK)], stage_bufs[slot], lsems[slot]))

        def _fire_cnt_load(blk):
            w0 = w_base + blk * WBLK
            c_pendl[blk % 2].append(pltpu.async_copy(
                cnt_sh.at[pl.ds(w0, WBLK)], cnt_bufs[blk % 2],
                clsems[blk % 2]))

        for blk in range(min(SSLOT - 1, N_WBLK)):
            _fire_stage_load(blk)
        for blk in range(min(2, N_WBLK)):
            _fire_cnt_load(blk)
        for blk in range(N_WBLK):
            slot = blk % SSLOT
            w0 = w_base + blk * WBLK
            for d in l_pend[slot]:
                d.wait()
            l_pend[slot] = []
            for d in c_pendl[blk % 2]:
                d.wait()
            c_pendl[blk % 2] = []
            z_pend.append(pltpu.async_copy(
                zbuf_v, acc_sh.at[pl.ds(w0, WBLK)], zsem))
            if last_slice:
                z_pend.append(pltpu.async_copy(
                    zbuf_v, cnt_sh.at[pl.ds(w0, WBLK)], csem))
            nxt = blk + SSLOT - 1
            if nxt < N_WBLK:
                for d in w_pend[nxt % SSLOT]:
                    d.wait()
                w_pend[nxt % SSLOT] = []
                _fire_stage_load(nxt)

            stage = stage_bufs[slot]
            cbase = (blk % 2) * WBLK

            def _div_row(i, _, stage=stage, cbase=cbase):
                cvec = cw_v[cbase + i, pl.ds(0, L)]
                recip = 1.0 / jnp.maximum(cvec, 1.0)
                for j in range(DSL // L):
                    stage[i, pl.ds(j * L, L)] = (
                        stage[i, pl.ds(j * L, L)] * recip)
                return 0
            lax.fori_loop(0, WBLK, _div_row, 0)

            if blk + 2 < N_WBLK:
                _fire_cnt_load(blk + 2)

            w_pend[slot].append(pltpu.async_copy(
                stage, out_hbm.at[b, pl.ds(w0, WBLK), pl.ds(doff, DSL)],
                wsems[slot]))

        for slot in range(SSLOT):
            for d in w_pend[slot]:
                d.wait()
        for d in z_pend:
            d.wait()

        plsc.subcore_barrier()


@jax.jit
def _pool(token_embeds, token_to_words):
    mesh = plsc.VectorSubcoreMesh(core_axis_name="c", subcore_axis_name="s",
                                  num_cores=NC, num_subcores=NS)
    kern = functools.partial(
        pl.kernel,
        out_type=jax.ShapeDtypeStruct((B, W, D), jnp.float32),
        mesh=mesh,
        scratch_types=[
            pltpu.VMEM((N_CHUNKS, CHUNK), jnp.int32),
            pltpu.VMEM((4, CHUNK, DSL), jnp.float32),
            pltpu.VMEM((4, WBLK, DSL), jnp.float32),
            pltpu.VMEM((CHUNK, DSL), jnp.float32),
            pltpu.VMEM((WBLK, DSL), jnp.float32),
            pltpu.VMEM_SHARED((W, DSL), jnp.float32),
            pltpu.VMEM_SHARED((W, DSL), jnp.float32),
            [pltpu.SemaphoreType.DMA] * 4,
            [pltpu.SemaphoreType.DMA] * 4,
            [pltpu.SemaphoreType.DMA] * 4,
            [pltpu.SemaphoreType.DMA] * 4,
            [pltpu.SemaphoreType.DMA] * 2,
            pltpu.SemaphoreType.DMA,
            pltpu.SemaphoreType.DMA,
        ],
    )(_pool_body)
    return kern(token_embeds, token_to_words)


def kernel(token_embeds, token_to_words):
    return _pool(token_embeds, token_to_words)


# speedup vs baseline: 1.0312x; 1.0312x over previous
import functools

import jax
import jax.numpy as jnp
from jax import lax
from jax.experimental import pallas as pl
from jax.experimental.pallas import tpu as pltpu
from jax.experimental.pallas import tpu_sc as plsc

B, S, D, W = 4, 8192, 768, 4096
NC, NS, L = 2, 16, 16
DSL = 128
N_SLICES = D // DSL
CHUNK = 64
TOK_PER_TILE = S // NS
N_CHUNKS = TOK_PER_TILE // CHUNK
W_PER_TILE = W // NS
WBLK = 32
N_WBLK = W_PER_TILE // WBLK
B_PER_SC = B // NC
N_ROUNDS = B_PER_SC * N_SLICES


NSLOT = 4


def _pool_body(emb_hbm, ids_hbm, out_hbm,
               ids_all_v, tok2_v, stage4_v, cw_v, zbuf_v,
               acc_sh, cnt_sh,
               gsems, ssems, lsems, wsems, clsems, zsem, csem):
    c = lax.axis_index("c")
    s = lax.axis_index("s")

    zvec = jnp.zeros((L,), jnp.float32)
    ovec = jnp.ones((L,), jnp.float32)

    def _init_row(i, _):
        for j in range(DSL // L):
            zbuf_v[i, pl.ds(j * L, L)] = zvec
        return 0
    lax.fori_loop(0, WBLK, _init_row, 0)

    w_base = s * W_PER_TILE

    zeros0 = []
    for blk in range(N_WBLK):
        zeros0.append(pltpu.async_copy(
            zbuf_v, acc_sh.at[pl.ds(w_base + blk * WBLK, WBLK)], zsem))
        zeros0.append(pltpu.async_copy(
            zbuf_v, cnt_sh.at[pl.ds(w_base + blk * WBLK, WBLK)], csem))
    for d in zeros0:
        d.wait()

    ids_rows = [ids_all_v.at[k] for k in range(N_CHUNKS)]
    tok_bufs = [tok2_v.at[i] for i in range(NSLOT)]
    g_pend = [[] for _ in range(NSLOT)]

    def _round_params(r):
        return c * B_PER_SC + r // N_SLICES, (r % N_SLICES) * DSL

    def _fire_gather(r, k):
        b_, doff_ = _round_params(r)
        slot = k % NSLOT
        t0 = s * TOK_PER_TILE + k * CHUNK
        g_pend[slot].append(pltpu.async_copy(
            emb_hbm.at[b_, pl.ds(t0, CHUNK), pl.ds(doff_, DSL)],
            tok_bufs[slot], gsems[slot]))

    for k in range(NSLOT):
        _fire_gather(0, k)

    plsc.subcore_barrier()

    for r in range(N_ROUNDS):
        b, doff = _round_params(r)
        dslice = r % N_SLICES
        first_slice = dslice == 0
        last_slice = dslice == N_SLICES - 1

        if first_slice:
            i_pend = []
            for k in range(N_CHUNKS):
                t0 = s * TOK_PER_TILE + k * CHUNK
                i_pend.append(pltpu.async_copy(
                    ids_hbm.at[b, pl.ds(t0, CHUNK)], ids_rows[k], csem))
            for d in i_pend:
                d.wait()
            def _init_ones(i, _):
                for j in range(DSL // L):
                    cw_v[i, pl.ds(j * L, L)] = ovec
                return 0
            lax.fori_loop(0, CHUNK, _init_ones, 0)

        s_pend = [[] for _ in range(NSLOT)]
        c_pend = []
        for k in range(N_CHUNKS):
            slot = k % NSLOT
            for d in g_pend[slot]:
                d.wait()
            g_pend[slot] = []
            s_pend[slot].append(pltpu.async_copy(
                tok_bufs[slot], acc_sh.at[ids_rows[k]], ssems[slot],
                add=True))
            if first_slice:
                c_pend.append(pltpu.async_copy(
                    cw_v, cnt_sh.at[ids_rows[k]], csem, add=True))
            nxt = k + NSLOT
            if nxt < N_CHUNKS:
                nslot = nxt % NSLOT
                for d in s_pend[nslot]:
                    d.wait()
                s_pend[nslot] = []
                _fire_gather(r, nxt)
        for slot in range(NSLOT):
            for d in s_pend[slot]:
                d.wait()
        for d in c_pend:
            d.wait()

        if r + 1 < N_ROUNDS:
            for k in range(NSLOT):
                _fire_gather(r + 1, k)

        plsc.subcore_barrier()

        SSLOT = 4
        stage_bufs = [stage4_v.at[i] for i in range(SSLOT)]
        cnt_bufs = [cw_v.at[pl.ds(0, WBLK)], cw_v.at[pl.ds(WBLK, WBLK)]]
        l_pend = [[] for _ in range(SSLOT)]
        c_pendl = [[], []]
        w_pend = [[] for _ in range(SSLOT)]
        z_pend = []

        def _fire_stage_load(blk):
            slot = blk % SSLOT
            w0 = w_base + blk * WBLK
            l_pend[slot].append(pltpu.async_copy(
                acc_sh.at[pl.ds(w0, WBLK)], stage_bufs[slot], lsems[slot]))

        def _fire_cnt_load(blk):
            w0 = w_base + blk * WBLK
            c_pendl[blk % 2].append(pltpu.async_copy(
                cnt_sh.at[pl.ds(w0, WBLK)], cnt_bufs[blk % 2],
                clsems[blk % 2]))

        for blk in range(min(SSLOT - 1, N_WBLK)):
            _fire_stage_load(blk)
        for blk in range(min(2, N_WBLK)):
            _fire_cnt_load(blk)
        for blk in range(N_WBLK):
            slot = blk % SSLOT
            w0 = w_base + blk * WBLK
            for d in l_pend[slot]:
                d.wait()
            l_pend[slot] = []
            for d in c_pendl[blk % 2]:
                d.wait()
            c_pendl[blk % 2] = []
            z_pend.append(pltpu.async_copy(
                zbuf_v, acc_sh.at[pl.ds(w0, WBLK)], zsem))
            if last_slice:
                z_pend.append(pltpu.async_copy(
                    zbuf_v, cnt_sh.at[pl.ds(w0, WBLK)], csem))
            nxt = blk + SSLOT - 1
            if nxt < N_WBLK:
                for d in w_pend[nxt % SSLOT]:
                    d.wait()
                w_pend[nxt % SSLOT] = []
                _fire_stage_load(nxt)

            stage = stage_bufs[slot]
            cbase = (blk % 2) * WBLK

            def _div_row(i, _, stage=stage, cbase=cbase):
                cvec = cw_v[cbase + i, pl.ds(0, L)]
                recip = 1.0 / jnp.maximum(cvec, 1.0)
                for j in range(DSL // L):
                    stage[i, pl.ds(j * L, L)] = (
                        stage[i, pl.ds(j * L, L)] * recip)
                return 0
            lax.fori_loop(0, WBLK, _div_row, 0)

            if blk + 2 < N_WBLK:
                _fire_cnt_load(blk + 2)

            w_pend[slot].append(pltpu.async_copy(
                stage, out_hbm.at[b, pl.ds(w0, WBLK), pl.ds(doff, DSL)],
                wsems[slot]))

        for slot in range(SSLOT):
            for d in w_pend[slot]:
                d.wait()
        for d in z_pend:
            d.wait()

        plsc.subcore_barrier()


@jax.jit
def _pool(token_embeds, token_to_words):
    mesh = plsc.VectorSubcoreMesh(core_axis_name="c", subcore_axis_name="s",
                                  num_cores=NC, num_subcores=NS)
    kern = functools.partial(
        pl.kernel,
        out_type=jax.ShapeDtypeStruct((B, W, D), jnp.float32),
        mesh=mesh,
        scratch_types=[
            pltpu.VMEM((N_CHUNKS, CHUNK), jnp.int32),
            pltpu.VMEM((4, CHUNK, DSL), jnp.float32),
            pltpu.VMEM((4, WBLK, DSL), jnp.float32),
            pltpu.VMEM((CHUNK, DSL), jnp.float32),
            pltpu.VMEM((WBLK, DSL), jnp.float32),
            pltpu.VMEM_SHARED((W, DSL), jnp.float32),
            pltpu.VMEM_SHARED((W, DSL), jnp.float32),
            [pltpu.SemaphoreType.DMA] * 4,
            [pltpu.SemaphoreType.DMA] * 4,
            [pltpu.SemaphoreType.DMA] * 4,
            [pltpu.SemaphoreType.DMA] * 4,
            [pltpu.SemaphoreType.DMA] * 2,
            pltpu.SemaphoreType.DMA,
            pltpu.SemaphoreType.DMA,
        ],
    )(_pool_body)
    return kern(token_embeds, token_to_words)


def kernel(token_embeds, token_to_words):
    return _pool(token_embeds, token_to_words)
